# baseline (speedup 1.0000x reference)
import functools

import jax
import jax.numpy as jnp
from jax import lax
from jax.experimental import pallas as pl
from jax.experimental.pallas import tpu as pltpu
from jax.experimental.pallas import tpu_sc as plsc

_N = 10000
_D = 128
_E = 320000
_NW = 32
_NTILE = 16
_CHUNK = 128
_NCH = 80
_NCH2 = _NCH // 2
_KPIPE = 2
_E_PAD = _NW * _NCH * _CHUNK
_N_ACC = 10112
_RPT = _N_ACC // _NTILE
_EPW = _NCH * _CHUNK
_AC = 5
_AR = _N_ACC // 128

_NCHH = _NCH // 2


def _attr_body(dst3b, attrt, zacc,
               ps,
               dst_v, *bufs):
    attr_vs = bufs[: _AC]
    acc_vs = bufs[_AC: 2 * _AC]
    c = lax.axis_index("c")
    s = lax.axis_index("s")
    w = c * _NTILE + s
    for k in range(_AC):
        pltpu.sync_copy(zacc, acc_vs[k])

    for half in range(2):
        pltpu.sync_copy(dst3b.at[w * 2 + half], dst_v)
        for k in range(_AC):
            pltpu.sync_copy(attrt.at[(w * _AC + k) * 2 + half], attr_vs[k])

        def chunk(i, carry):
            for j in range(_CHUNK // 16):
                dst16 = dst_v[i, pl.ds(j * 16, 16)]
                hi = lax.shift_right_logical(dst16, 7)
                lo = lax.bitwise_and(dst16, 127)
                for k in range(_AC):
                    v16 = attr_vs[k][i, pl.ds(j * 16, 16)]
                    plsc.addupdate_scatter(acc_vs[k], [hi, lo], v16)
            return carry

        lax.fori_loop(0, _NCHH, chunk, 0)
    for k in range(_AC):
        pltpu.sync_copy(acc_vs[k], ps.at[w * _AC + k])


@functools.cache
def _get_attr():
    return pl.kernel(
        _attr_body,
        out_type=jax.ShapeDtypeStruct((_NW * _AC, _AR, 128), jnp.float32),
        mesh=plsc.VectorSubcoreMesh(core_axis_name="c", subcore_axis_name="s"),
        compiler_params=pltpu.CompilerParams(needs_layout_passes=False),
        scratch_types=[pltpu.VMEM((_NCHH, _CHUNK), jnp.int32)]
        + [pltpu.VMEM((_NCHH, _CHUNK), jnp.float32) for _ in range(_AC)]
        + [pltpu.VMEM((_AR, 128), jnp.float32) for _ in range(_AC)],
    )


def _spmm_body(table, src3, dst3,
               px,
               src_v, dst_v, rows_a, xacc, sem_a):
    c = lax.axis_index("c")
    s = lax.axis_index("s")
    w = c * _NTILE + s
    row0 = s * _RPT

    def zrow(r, carry):
        for j in range(_D // 16):
            rows_a[r, pl.ds(j * 16, 16)] = jnp.zeros((16,), jnp.float32)
        return carry

    lax.fori_loop(0, _CHUNK, zrow, 0)
    nfull, rem = _RPT // _CHUNK, _RPT % _CHUNK
    zd = [pltpu.async_copy(rows_a,
                           xacc.at[pl.ds(row0 + k * _CHUNK, _CHUNK)], sem_a)
          for k in range(nfull)]
    if rem:
        zd.append(pltpu.async_copy(
            rows_a.at[pl.ds(0, rem)],
            xacc.at[pl.ds(row0 + nfull * _CHUNK, rem)], sem_a))
    for d in zd:
        d.wait()
    plsc.subcore_barrier()

    for half in range(2):
        pltpu.sync_copy(src3.at[w * 2 + half], src_v)
        pltpu.sync_copy(dst3.at[w * 2 + half], dst_v)

        def chunk(i, carry):
            pltpu.async_copy(table.at[src_v.at[i]], rows_a, sem_a).wait()
            pltpu.sync_copy(rows_a, xacc.at[dst_v.at[i]], add=True)
            return carry

        lax.fori_loop(0, _NCH2, chunk, 0)
    plsc.subcore_barrier()
    pltpu.sync_copy(xacc.at[pl.ds(row0, _RPT)], px.at[c, pl.ds(row0, _RPT)])


@functools.cache
def _get_spmm():
    return pl.kernel(
        _spmm_body,
        out_type=jax.ShapeDtypeStruct((2, _N_ACC, _D), jnp.float32),
        mesh=plsc.VectorSubcoreMesh(core_axis_name="c", subcore_axis_name="s"),
        scratch_types=[
            pltpu.VMEM((_NCH2, _CHUNK), jnp.int32),
            pltpu.VMEM((_NCH2, _CHUNK), jnp.int32),
            pltpu.VMEM((_CHUNK, _D), jnp.float32),
            pltpu.VMEM_SHARED((_N_ACC, _D), jnp.float32),
            pltpu.SemaphoreType.DMA,
        ],
    )


def _mlp0_body(px, ps, x, sl5, encw5, w1, b1, g, be, w2, b2, out):
    p = px[0, : _N, :] + px[1, : _N, :] + x[...]
    st = jnp.sum(ps[...], axis=0)[:, : _N] + sl5[...]
    agg = p + lax.dot_general(st, encw5[...], (((0,), (0,)), ((), ())),
                              preferred_element_type=jnp.float32)
    z = jnp.dot(agg, w1[...], preferred_element_type=jnp.float32) + b1[...]
    mu = jnp.mean(z, axis=0, keepdims=True)
    var = jnp.mean((z - mu) * (z - mu), axis=0, keepdims=True)
    z = (z - mu) * lax.rsqrt(var + 1e-5) * g[...] + be[...]
    z = jnp.maximum(z, 0.0)
    h = jnp.dot(z, w2[...], preferred_element_type=jnp.float32) + b2[...]
    out[...] = jnp.maximum(h, 0.0)


def _mlp1_body(ph, ps, h1, sl5, encw5, w1a, w1b, b1, g, be, w2, b2, out):
    p = ph[0, : _N, :] + ph[1, : _N, :] + h1[...]
    st = jnp.sum(ps[...], axis=0)[:, : _N] + sl5[...]
    aggr = lax.dot_general(st, encw5[...], (((0,), (0,)), ((), ())),
                           preferred_element_type=jnp.float32)
    z = (jnp.dot(p, w1a[...], preferred_element_type=jnp.float32)
         + jnp.dot(aggr, w1b[...], preferred_element_type=jnp.float32)
         + b1[...])
    mu = jnp.mean(z, axis=0, keepdims=True)
    var = jnp.mean((z - mu) * (z - mu), axis=0, keepdims=True)
    z = (z - mu) * lax.rsqrt(var + 1e-5) * g[...] + be[...]
    z = jnp.maximum(z, 0.0)
    out[...] = jnp.dot(z, w2[...], preferred_element_type=jnp.float32) + b2[...]


def kernel(x, edge_index, edge_attr, self_loop_index, self_loop_type,
           encW0, encb0, W10, b10, g0, be0, W20, b20,
           encW1, encb1, W11, b11, g1, be1, W21, b21):
    dst = edge_index[0]
    src = edge_index[1]
    pad = _E_PAD - _E
    src3 = jnp.concatenate(
        [src, jnp.zeros((pad,), jnp.int32)]).reshape(_NW * 2, _NCH2, _CHUNK)
    dst_p = jnp.concatenate([dst, jnp.full((pad,), _N, jnp.int32)])
    dst3 = dst_p.reshape(_NW * 2, _NCH2, _CHUNK)
    attrt = jnp.pad(
        jnp.concatenate([edge_attr, jnp.ones((_E, 1), jnp.float32)], axis=1),
        ((0, pad), (0, 0))).reshape(_NW, _EPW, _AC).transpose(0, 2, 1) \
        .reshape(_NW * _AC * 2, _NCHH, _CHUNK)
    zacc = jnp.zeros((_AR, 128), jnp.float32)

    ar5 = jnp.arange(_AC)
    sl5 = (jnp.where(ar5 == self_loop_index,
                     jnp.asarray(self_loop_type, jnp.float32), 0.0)
           + jnp.where(ar5 == _AC - 1, 1.0, 0.0)).reshape(_AC, 1)
    encw5_0 = jnp.concatenate([encW0, encb0[None, :]], axis=0)
    encw5_1 = jnp.concatenate([encW1, encb1[None, :]], axis=0)

    px = _get_spmm()(x, src3, dst3)
    ps = _get_attr()(dst3, attrt, zacc).reshape(_NW, _AC, _N_ACC)

    h1 = pl.pallas_call(
        _mlp0_body,
        out_shape=jax.ShapeDtypeStruct((_N, _D), jnp.float32),
    )(px, ps, x, sl5, encw5_0, W10, b10[None], g0[None], be0[None],
      W20, b20[None])

    ph = _get_spmm()(h1, src3, dst3)

    out = pl.pallas_call(
        _mlp1_body,
        out_shape=jax.ShapeDtypeStruct((_N, _D), jnp.float32),
    )(ph, ps, h1, sl5, encw5_1, W11[: _D], W11[_D:], b11[None], g1[None],
      be1[None], W21, b21[None])
    return out


# speedup vs baseline: 2.3639x; 2.3639x over previous
import functools

import jax
import jax.numpy as jnp
from jax import lax
from jax.experimental import pallas as pl
from jax.experimental.pallas import tpu as pltpu
from jax.experimental.pallas import tpu_sc as plsc

_N = 10000
_D = 128
_E = 320000
_NW = 32
_NTILE = 16
_CHUNK = 128
_NCH = 80
_NCH2 = _NCH // 2
_KPIPE = 2
_E_PAD = _NW * _NCH * _CHUNK
_N_ACC = 10112
_RPT = _N_ACC // _NTILE
_EPW = _NCH * _CHUNK
_AC = 5
_AR = _N_ACC // 128

_NCHH = _NCH // 2


def _attr_body(dst3b, attrt, zacc,
               ps,
               dst_v, *bufs):
    attr_vs = bufs[: _AC]
    acc_vs = bufs[_AC: 2 * _AC]
    c = lax.axis_index("c")
    s = lax.axis_index("s")
    w = c * _NTILE + s
    for k in range(_AC):
        pltpu.sync_copy(zacc, acc_vs[k])

    for half in range(2):
        pltpu.sync_copy(dst3b.at[w * 2 + half], dst_v)
        for k in range(_AC):
            pltpu.sync_copy(attrt.at[(w * _AC + k) * 2 + half], attr_vs[k])

        def chunk(i, carry):
            for j in range(_CHUNK // 16):
                dst16 = dst_v[i, pl.ds(j * 16, 16)]
                hi = lax.shift_right_logical(dst16, 7)
                lo = lax.bitwise_and(dst16, 127)
                for k in range(_AC):
                    v16 = attr_vs[k][i, pl.ds(j * 16, 16)]
                    plsc.addupdate_scatter(acc_vs[k], [hi, lo], v16)
            return carry

        lax.fori_loop(0, _NCHH, chunk, 0)
    for k in range(_AC):
        pltpu.sync_copy(acc_vs[k], ps.at[w * _AC + k])


@functools.cache
def _get_attr():
    return pl.kernel(
        _attr_body,
        out_type=jax.ShapeDtypeStruct((_NW * _AC, _AR, 128), jnp.float32),
        mesh=plsc.VectorSubcoreMesh(core_axis_name="c", subcore_axis_name="s"),
        compiler_params=pltpu.CompilerParams(needs_layout_passes=False),
        scratch_types=[pltpu.VMEM((_NCHH, _CHUNK), jnp.int32)]
        + [pltpu.VMEM((_NCHH, _CHUNK), jnp.float32) for _ in range(_AC)]
        + [pltpu.VMEM((_AR, 128), jnp.float32) for _ in range(_AC)],
    )


def _spmm_body(table, src3, dst3,
               px,
               src_v, dst_v, rows_a, xacc, sem_a):
    c = lax.axis_index("c")
    s = lax.axis_index("s")
    w = c * _NTILE + s
    row0 = s * _RPT

    def zrow(r, carry):
        for j in range(_D // 16):
            rows_a[r, pl.ds(j * 16, 16)] = jnp.zeros((16,), jnp.float32)
        return carry

    lax.fori_loop(0, _CHUNK, zrow, 0)
    nfull, rem = _RPT // _CHUNK, _RPT % _CHUNK
    zd = [pltpu.async_copy(rows_a,
                           xacc.at[pl.ds(row0 + k * _CHUNK, _CHUNK)], sem_a)
          for k in range(nfull)]
    if rem:
        zd.append(pltpu.async_copy(
            rows_a.at[pl.ds(0, rem)],
            xacc.at[pl.ds(row0 + nfull * _CHUNK, rem)], sem_a))
    for d in zd:
        d.wait()
    plsc.subcore_barrier()

    for half in range(2):
        pltpu.sync_copy(src3.at[w * 2 + half], src_v)
        pltpu.sync_copy(dst3.at[w * 2 + half], dst_v)

        def chunk(i, carry):
            pltpu.async_copy(table.at[src_v.at[i]], rows_a, sem_a).wait()
            pltpu.sync_copy(rows_a, xacc.at[dst_v.at[i]], add=True)
            return carry

        lax.fori_loop(0, _NCH2, chunk, 0)
    plsc.subcore_barrier()
    pltpu.sync_copy(xacc.at[pl.ds(row0, _RPT)], px.at[c, pl.ds(row0, _RPT)])


@functools.cache
def _get_spmm():
    return pl.kernel(
        _spmm_body,
        out_type=jax.ShapeDtypeStruct((2, _N_ACC, _D), jnp.float32),
        mesh=plsc.VectorSubcoreMesh(core_axis_name="c", subcore_axis_name="s"),
        scratch_types=[
            pltpu.VMEM((_NCH2, _CHUNK), jnp.int32),
            pltpu.VMEM((_NCH2, _CHUNK), jnp.int32),
            pltpu.VMEM((_CHUNK, _D), jnp.float32),
            pltpu.VMEM_SHARED((_N_ACC, _D), jnp.float32),
            pltpu.SemaphoreType.DMA,
        ],
    )


def _mlp0_body(px, ps, x, sl5, encw5, w1, b1, g, be, w2, b2, out):
    p = px[0, : _N, :] + px[1, : _N, :] + x[...]
    st = jnp.sum(ps[...], axis=0)[:, : _N] + sl5[...]
    agg = p + lax.dot_general(st, encw5[...], (((0,), (0,)), ((), ())),
                              preferred_element_type=jnp.float32)
    z = jnp.dot(agg, w1[...], preferred_element_type=jnp.float32) + b1[...]
    mu = jnp.mean(z, axis=0, keepdims=True)
    var = jnp.mean((z - mu) * (z - mu), axis=0, keepdims=True)
    z = (z - mu) * lax.rsqrt(var + 1e-5) * g[...] + be[...]
    z = jnp.maximum(z, 0.0)
    h = jnp.dot(z, w2[...], preferred_element_type=jnp.float32) + b2[...]
    out[...] = jnp.maximum(h, 0.0)


def _mlp1_body(ph, ps, h1, sl5, encw5, w1a, w1b, b1, g, be, w2, b2, out):
    p = ph[0, : _N, :] + ph[1, : _N, :] + h1[...]
    st = jnp.sum(ps[...], axis=0)[:, : _N] + sl5[...]
    aggr = lax.dot_general(st, encw5[...], (((0,), (0,)), ((), ())),
                           preferred_element_type=jnp.float32)
    z = (jnp.dot(p, w1a[...], preferred_element_type=jnp.float32)
         + jnp.dot(aggr, w1b[...], preferred_element_type=jnp.float32)
         + b1[...])
    mu = jnp.mean(z, axis=0, keepdims=True)
    var = jnp.mean((z - mu) * (z - mu), axis=0, keepdims=True)
    z = (z - mu) * lax.rsqrt(var + 1e-5) * g[...] + be[...]
    z = jnp.maximum(z, 0.0)
    out[...] = jnp.dot(z, w2[...], preferred_element_type=jnp.float32) + b2[...]


def kernel(x, edge_index, edge_attr, self_loop_index, self_loop_type,
           encW0, encb0, W10, b10, g0, be0, W20, b20,
           encW1, encb1, W11, b11, g1, be1, W21, b21):
    dst = edge_index[0]
    src = edge_index[1]
    pad = _E_PAD - _E
    padr = jnp.arange(pad, dtype=jnp.int32)
    src3 = jnp.concatenate(
        [src, padr % _N]).reshape(_NW * 2, _NCH2, _CHUNK)
    dst_p = jnp.concatenate([dst, _N + padr % (_N_ACC - _N)])
    dst3 = dst_p.reshape(_NW * 2, _NCH2, _CHUNK)
    attrt = jnp.pad(
        jnp.concatenate([edge_attr, jnp.ones((_E, 1), jnp.float32)], axis=1),
        ((0, pad), (0, 0))).reshape(_NW, _EPW, _AC).transpose(0, 2, 1) \
        .reshape(_NW * _AC * 2, _NCHH, _CHUNK)
    zacc = jnp.zeros((_AR, 128), jnp.float32)

    ar5 = jnp.arange(_AC)
    sl5 = (jnp.where(ar5 == self_loop_index,
                     jnp.asarray(self_loop_type, jnp.float32), 0.0)
           + jnp.where(ar5 == _AC - 1, 1.0, 0.0)).reshape(_AC, 1)
    encw5_0 = jnp.concatenate([encW0, encb0[None, :]], axis=0)
    encw5_1 = jnp.concatenate([encW1, encb1[None, :]], axis=0)

    px = _get_spmm()(x, src3, dst3)
    ps = _get_attr()(dst3, attrt, zacc).reshape(_NW, _AC, _N_ACC)

    h1 = pl.pallas_call(
        _mlp0_body,
        out_shape=jax.ShapeDtypeStruct((_N, _D), jnp.float32),
    )(px, ps, x, sl5, encw5_0, W10, b10[None], g0[None], be0[None],
      W20, b20[None])

    ph = _get_spmm()(h1, src3, dst3)

    out = pl.pallas_call(
        _mlp1_body,
        out_shape=jax.ShapeDtypeStruct((_N, _D), jnp.float32),
    )(ph, ps, h1, sl5, encw5_1, W11[: _D], W11[_D:], b11[None], g1[None],
      be1[None], W21, b21[None])
    return out


# speedup vs baseline: 2.6540x; 1.1227x over previous
import functools

import jax
import jax.numpy as jnp
from jax import lax
from jax.experimental import pallas as pl
from jax.experimental.pallas import tpu as pltpu
from jax.experimental.pallas import tpu_sc as plsc

_N = 10000
_D = 128
_E = 320000
_NW = 32
_NTILE = 16
_CHUNK = 128
_NCH = 80
_NCH2 = _NCH // 2
_KPIPE = 2
_E_PAD = _NW * _NCH * _CHUNK
_N_ACC = 10112
_RPT = _N_ACC // _NTILE
_EPW = _NCH * _CHUNK
_AC = 5
_AR = _N_ACC // 128

_NCHH = _NCH // 2


def _attr_body(dst3b, attrt, zacc,
               ps,
               dst_v, *bufs):
    attr_vs = bufs[: _AC]
    acc_vs = bufs[_AC: 2 * _AC]
    c = lax.axis_index("c")
    s = lax.axis_index("s")
    w = c * _NTILE + s
    for k in range(_AC):
        pltpu.sync_copy(zacc, acc_vs[k])

    for half in range(2):
        pltpu.sync_copy(dst3b.at[w * 2 + half], dst_v)
        for k in range(_AC):
            pltpu.sync_copy(attrt.at[(w * _AC + k) * 2 + half], attr_vs[k])

        def chunk(i, carry):
            for j in range(_CHUNK // 16):
                dst16 = dst_v[i, pl.ds(j * 16, 16)]
                hi = lax.shift_right_logical(dst16, 7)
                lo = lax.bitwise_and(dst16, 127)
                for k in range(_AC):
                    v16 = attr_vs[k][i, pl.ds(j * 16, 16)]
                    plsc.addupdate_scatter(acc_vs[k], [hi, lo], v16)
            return carry

        lax.fori_loop(0, _NCHH, chunk, 0)
    for k in range(_AC):
        pltpu.sync_copy(acc_vs[k], ps.at[w * _AC + k])


@functools.cache
def _get_attr():
    return pl.kernel(
        _attr_body,
        out_type=jax.ShapeDtypeStruct((_NW * _AC, _AR, 128), jnp.float32),
        mesh=plsc.VectorSubcoreMesh(core_axis_name="c", subcore_axis_name="s"),
        compiler_params=pltpu.CompilerParams(needs_layout_passes=False),
        scratch_types=[pltpu.VMEM((_NCHH, _CHUNK), jnp.int32)]
        + [pltpu.VMEM((_NCHH, _CHUNK), jnp.float32) for _ in range(_AC)]
        + [pltpu.VMEM((_AR, 128), jnp.float32) for _ in range(_AC)],
    )


def _spmm_body(table, src3, dst3,
               px,
               src_v, dst_v, rows_a, rows_b, xacc, sem_a, sem_b):
    c = lax.axis_index("c")
    s = lax.axis_index("s")
    w = c * _NTILE + s
    row0 = s * _RPT

    def zrow(r, carry):
        for j in range(_D // 16):
            rows_a[r, pl.ds(j * 16, 16)] = jnp.zeros((16,), jnp.float32)
        return carry

    lax.fori_loop(0, _CHUNK, zrow, 0)
    nfull, rem = _RPT // _CHUNK, _RPT % _CHUNK
    zd = [pltpu.async_copy(rows_a,
                           xacc.at[pl.ds(row0 + k * _CHUNK, _CHUNK)], sem_a)
          for k in range(nfull)]
    if rem:
        zd.append(pltpu.async_copy(
            rows_a.at[pl.ds(0, rem)],
            xacc.at[pl.ds(row0 + nfull * _CHUNK, rem)], sem_a))
    for d in zd:
        d.wait()
    plsc.subcore_barrier()

    for half in range(2):
        pltpu.sync_copy(src3.at[w * 2 + half], src_v)
        pltpu.sync_copy(dst3.at[w * 2 + half], dst_v)

        def chunk(t, carry):
            i0 = 2 * t
            da = pltpu.async_copy(table.at[src_v.at[i0]], rows_a, sem_a)
            db = pltpu.async_copy(table.at[src_v.at[i0 + 1]], rows_b, sem_b)
            da.wait()
            pltpu.sync_copy(rows_a, xacc.at[dst_v.at[i0]], add=True)
            db.wait()
            pltpu.sync_copy(rows_b, xacc.at[dst_v.at[i0 + 1]], add=True)
            return carry

        lax.fori_loop(0, _NCH2 // 2, chunk, 0)
    plsc.subcore_barrier()
    pltpu.sync_copy(xacc.at[pl.ds(row0, _RPT)], px.at[c, pl.ds(row0, _RPT)])


@functools.cache
def _get_spmm():
    return pl.kernel(
        _spmm_body,
        out_type=jax.ShapeDtypeStruct((2, _N_ACC, _D), jnp.float32),
        mesh=plsc.VectorSubcoreMesh(core_axis_name="c", subcore_axis_name="s"),
        scratch_types=[
            pltpu.VMEM((_NCH2, _CHUNK), jnp.int32),
            pltpu.VMEM((_NCH2, _CHUNK), jnp.int32),
            pltpu.VMEM((_CHUNK, _D), jnp.float32),
            pltpu.VMEM((_CHUNK, _D), jnp.float32),
            pltpu.VMEM_SHARED((_N_ACC, _D), jnp.float32),
            pltpu.SemaphoreType.DMA,
            pltpu.SemaphoreType.DMA,
        ],
    )


def _mlp0_body(px, ps, x, sl5, encw5, w1, b1, g, be, w2, b2, out):
    p = px[0, : _N, :] + px[1, : _N, :] + x[...]
    st = jnp.sum(ps[...], axis=0)[:, : _N] + sl5[...]
    agg = p + lax.dot_general(st, encw5[...], (((0,), (0,)), ((), ())),
                              preferred_element_type=jnp.float32)
    z = jnp.dot(agg, w1[...], preferred_element_type=jnp.float32) + b1[...]
    mu = jnp.mean(z, axis=0, keepdims=True)
    var = jnp.mean((z - mu) * (z - mu), axis=0, keepdims=True)
    z = (z - mu) * lax.rsqrt(var + 1e-5) * g[...] + be[...]
    z = jnp.maximum(z, 0.0)
    h = jnp.dot(z, w2[...], preferred_element_type=jnp.float32) + b2[...]
    out[...] = jnp.maximum(h, 0.0)


def _mlp1_body(ph, ps, h1, sl5, encw5, w1a, w1b, b1, g, be, w2, b2, out):
    p = ph[0, : _N, :] + ph[1, : _N, :] + h1[...]
    st = jnp.sum(ps[...], axis=0)[:, : _N] + sl5[...]
    aggr = lax.dot_general(st, encw5[...], (((0,), (0,)), ((), ())),
                           preferred_element_type=jnp.float32)
    z = (jnp.dot(p, w1a[...], preferred_element_type=jnp.float32)
         + jnp.dot(aggr, w1b[...], preferred_element_type=jnp.float32)
         + b1[...])
    mu = jnp.mean(z, axis=0, keepdims=True)
    var = jnp.mean((z - mu) * (z - mu), axis=0, keepdims=True)
    z = (z - mu) * lax.rsqrt(var + 1e-5) * g[...] + be[...]
    z = jnp.maximum(z, 0.0)
    out[...] = jnp.dot(z, w2[...], preferred_element_type=jnp.float32) + b2[...]


def kernel(x, edge_index, edge_attr, self_loop_index, self_loop_type,
           encW0, encb0, W10, b10, g0, be0, W20, b20,
           encW1, encb1, W11, b11, g1, be1, W21, b21):
    dst = edge_index[0]
    src = edge_index[1]
    pad = _E_PAD - _E
    padr = jnp.arange(pad, dtype=jnp.int32)
    src3 = jnp.concatenate(
        [src, padr % _N]).reshape(_NW * 2, _NCH2, _CHUNK)
    dst_p = jnp.concatenate([dst, _N + padr % (_N_ACC - _N)])
    dst3 = dst_p.reshape(_NW * 2, _NCH2, _CHUNK)
    attrt = jnp.pad(
        jnp.concatenate([edge_attr, jnp.ones((_E, 1), jnp.float32)], axis=1),
        ((0, pad), (0, 0))).reshape(_NW, _EPW, _AC).transpose(0, 2, 1) \
        .reshape(_NW * _AC * 2, _NCHH, _CHUNK)
    zacc = jnp.zeros((_AR, 128), jnp.float32)

    ar5 = jnp.arange(_AC)
    sl5 = (jnp.where(ar5 == self_loop_index,
                     jnp.asarray(self_loop_type, jnp.float32), 0.0)
           + jnp.where(ar5 == _AC - 1, 1.0, 0.0)).reshape(_AC, 1)
    encw5_0 = jnp.concatenate([encW0, encb0[None, :]], axis=0)
    encw5_1 = jnp.concatenate([encW1, encb1[None, :]], axis=0)

    px = _get_spmm()(x, src3, dst3)
    ps = _get_attr()(dst3, attrt, zacc).reshape(_NW, _AC, _N_ACC)

    h1 = pl.pallas_call(
        _mlp0_body,
        out_shape=jax.ShapeDtypeStruct((_N, _D), jnp.float32),
    )(px, ps, x, sl5, encw5_0, W10, b10[None], g0[None], be0[None],
      W20, b20[None])

    ph = _get_spmm()(h1, src3, dst3)

    out = pl.pallas_call(
        _mlp1_body,
        out_shape=jax.ShapeDtypeStruct((_N, _D), jnp.float32),
    )(ph, ps, h1, sl5, encw5_1, W11[: _D], W11[_D:], b11[None], g1[None],
      be1[None], W21, b21[None])
    return out
